# baseline (speedup 1.0000x reference)
import jax
import jax.numpy as jnp
from jax.experimental import pallas as pl
from jax.experimental.pallas import tpu as pltpu


def _body(dq_ref, x_ref, w1_ref, b1_ref, w2_ref, b2_ref, emb_ref,
          zq_ref, ind_ref, diff_ref, acc_ref):
    i = pl.program_id(0)
    nb = pl.num_programs(0)
    bm = x_ref.shape[0]
    ncodes = emb_ref.shape[1]
    dm = emb_ref.shape[0]

    h = jnp.dot(x_ref[...], w1_ref[...]) + b1_ref[...]
    h = jnp.where(h >= 0, h, 0.01 * h)
    z = jnp.dot(h, w2_ref[...]) + b2_ref[...]

    emb = emb_ref[...]
    zsq = (z ** 2).sum(axis=1, keepdims=True)
    esq = (emb ** 2).sum(axis=0, keepdims=True)
    dist = zsq - 2.0 * jnp.dot(z, emb) + esq

    minval = jnp.min(dist, axis=1, keepdims=True)
    iota = jax.lax.broadcasted_iota(jnp.int32, (bm, ncodes), 1)
    ind = jnp.min(jnp.where(dist == minval, iota, ncodes), axis=1)

    onehot = (iota == ind[:, None]).astype(jnp.float32)
    q = jax.lax.dot_general(onehot, emb, (((1,), (1,)), ((), ())))

    dq = dq_ref[0] != 0
    zq_ref[...] = jnp.where(dq, q, z)
    ind_ref[...] = ind.reshape(1, 1, bm)

    d = q - z
    psum = jnp.sum(d * d)

    @pl.when(i == 0)
    def _init():
        acc_ref[0] = 0.0

    acc_ref[0] += psum

    @pl.when(i == nb - 1)
    def _fin():
        diff_ref[0, 0] = jnp.where(dq, acc_ref[0] / (nb * bm * dm), 0.0)


def _encode(dq, x, w1, b1, w2, b2, emb, *, bm=128, interpret=False):
    b, inp = x.shape
    dh = w1.shape[1]
    dm, ncodes = emb.shape
    nb = b // bm
    zq, ind, diff = pl.pallas_call(
        _body,
        grid=(nb,),
        in_specs=[
            pl.BlockSpec(memory_space=pltpu.SMEM),
            pl.BlockSpec((bm, inp), lambda i: (i, 0)),
            pl.BlockSpec((inp, dh), lambda i: (0, 0)),
            pl.BlockSpec((1, dh), lambda i: (0, 0)),
            pl.BlockSpec((dh, dm), lambda i: (0, 0)),
            pl.BlockSpec((1, dm), lambda i: (0, 0)),
            pl.BlockSpec((dm, ncodes), lambda i: (0, 0)),
        ],
        out_specs=[
            pl.BlockSpec((bm, dm), lambda i: (i, 0)),
            pl.BlockSpec((1, 1, bm), lambda i: (i, 0, 0)),
            pl.BlockSpec(memory_space=pltpu.SMEM),
        ],
        out_shape=[
            jax.ShapeDtypeStruct((b, dm), jnp.float32),
            jax.ShapeDtypeStruct((nb, 1, bm), jnp.int32),
            jax.ShapeDtypeStruct((1, 1), jnp.float32),
        ],
        scratch_shapes=[pltpu.SMEM((1,), jnp.float32)],
        compiler_params=pltpu.CompilerParams(
            dimension_semantics=("arbitrary",),
        ),
        interpret=interpret,
    )(dq, x, w1, b1, w2, b2, emb)
    return zq, ind, diff


def kernel(x, W1, b1, W2, b2, embed, do_quantize, k):
    b = x.shape[0]
    xin = x.reshape((b, -1))
    dq = jnp.asarray(do_quantize, jnp.int32).reshape(1)
    zq, ind, diff = _encode(
        dq, xin, W1, b1.reshape(1, -1), W2, b2.reshape(1, -1), embed)
    return zq, diff.reshape(()), ind.reshape(1, b)


# speedup vs baseline: 1.3621x; 1.3621x over previous
import jax
import jax.numpy as jnp
from jax.experimental import pallas as pl
from jax.experimental.pallas import tpu as pltpu


def _body(dq_ref, x_ref, w1_ref, b1_ref, w2_ref, b2_ref, emb_ref,
          zq_ref, ind_ref, diff_ref, acc_ref):
    i = pl.program_id(0)
    nb = pl.num_programs(0)
    bm = x_ref.shape[0]
    ncodes = emb_ref.shape[1]
    dm = emb_ref.shape[0]

    h = jnp.dot(x_ref[...], w1_ref[...]) + b1_ref[...]
    h = jnp.where(h >= 0, h, 0.01 * h)
    z = jnp.dot(h, w2_ref[...]) + b2_ref[...]

    emb = emb_ref[...]
    zsq = (z ** 2).sum(axis=1, keepdims=True)
    esq = (emb ** 2).sum(axis=0, keepdims=True)
    dist = zsq - 2.0 * jnp.dot(z, emb) + esq

    minval = jnp.min(dist, axis=1, keepdims=True)
    iota = jax.lax.broadcasted_iota(jnp.int32, (bm, ncodes), 1)
    ind = jnp.min(jnp.where(dist == minval, iota, ncodes), axis=1)

    onehot = (iota == ind[:, None]).astype(jnp.float32)
    q = jax.lax.dot_general(onehot, emb, (((1,), (1,)), ((), ())))

    dq = dq_ref[0] != 0
    zq_ref[...] = jnp.where(dq, q, z)
    ind_ref[...] = ind.reshape(1, 1, bm)

    d = q - z
    psum = jnp.sum(d * d)

    @pl.when(i == 0)
    def _init():
        acc_ref[0] = 0.0

    acc_ref[0] += psum

    @pl.when(i == nb - 1)
    def _fin():
        diff_ref[0, 0] = jnp.where(dq, acc_ref[0] / (nb * bm * dm), 0.0)


def _encode(dq, x, w1, b1, w2, b2, emb, *, bm=512, interpret=False):
    b, inp = x.shape
    dh = w1.shape[1]
    dm, ncodes = emb.shape
    nb = b // bm
    zq, ind, diff = pl.pallas_call(
        _body,
        grid=(nb,),
        in_specs=[
            pl.BlockSpec(memory_space=pltpu.SMEM),
            pl.BlockSpec((bm, inp), lambda i: (i, 0)),
            pl.BlockSpec((inp, dh), lambda i: (0, 0)),
            pl.BlockSpec((1, dh), lambda i: (0, 0)),
            pl.BlockSpec((dh, dm), lambda i: (0, 0)),
            pl.BlockSpec((1, dm), lambda i: (0, 0)),
            pl.BlockSpec((dm, ncodes), lambda i: (0, 0)),
        ],
        out_specs=[
            pl.BlockSpec((bm, dm), lambda i: (i, 0)),
            pl.BlockSpec((1, 1, bm), lambda i: (i, 0, 0)),
            pl.BlockSpec(memory_space=pltpu.SMEM),
        ],
        out_shape=[
            jax.ShapeDtypeStruct((b, dm), jnp.float32),
            jax.ShapeDtypeStruct((nb, 1, bm), jnp.int32),
            jax.ShapeDtypeStruct((1, 1), jnp.float32),
        ],
        scratch_shapes=[pltpu.SMEM((1,), jnp.float32)],
        compiler_params=pltpu.CompilerParams(
            dimension_semantics=("arbitrary",),
        ),
        interpret=interpret,
    )(dq, x, w1, b1, w2, b2, emb)
    return zq, ind, diff


def kernel(x, W1, b1, W2, b2, embed, do_quantize, k):
    b = x.shape[0]
    xin = x.reshape((b, -1))
    dq = jnp.asarray(do_quantize, jnp.int32).reshape(1)
    zq, ind, diff = _encode(
        dq, xin, W1, b1.reshape(1, -1), W2, b2.reshape(1, -1), embed)
    return zq, diff.reshape(()), ind.reshape(1, b)
